# baseline (speedup 1.0000x reference)
import jax
import jax.numpy as jnp
from jax import lax
from jax.experimental import pallas as pl
from jax.experimental.pallas import tpu as pltpu
from jax.experimental.pallas import tpu_sc as plsc

P = 128
K = 8
L = 16
CHUNKS = P // L
BIG = 3.0e38
IDX_MASK = 0x7F
NINTH = 1.0 / 9.0


def _sc_body(d_hbm, out_hbm, in_v0, in_v1, out_v0, out_v1,
             isem0, isem1, osem0, osem1):
    num_cores = 2
    wid = lax.axis_index("s") * num_cores + lax.axis_index("c")

    lane = lax.iota(jnp.int32, L)
    m_lo = lane < K
    m_sel = lane <= K
    chunk_idx = [lane + L * c for c in range(CHUNKS)]
    zeros16 = jnp.zeros((L,), jnp.float32)
    ninth16 = jnp.full((L,), NINTH, jnp.float32)
    big16 = jnp.full((L,), BIG, jnp.float32)
    hi_mask = jnp.full((L,), ~IDX_MASK, jnp.int32)
    lo_mask = jnp.full((L,), IDX_MASK, jnp.int32)

    def merge(a, b, out_inverted):
        ak, av = a
        bk, bv = b
        if out_inverted:
            ck = jnp.where(m_lo, -ak, bk)
        else:
            ck = jnp.where(m_lo, ak, -bk)
        cv = jnp.where(m_lo, av, bv)
        return plsc.sort_key_val(ck, cv)

    def compute_block(in_v, off, out_v):
        for g in range(CHUNKS):
            diag = lane + L * g
            plsc.store_scatter(in_v, [off + diag, diag], big16)

        def row_body(r):
            i_splat = jnp.full((L,), r, jnp.int32)
            level = []
            for c in range(CHUNKS):
                vals = in_v[off + r, pl.ds(c * L, L)]
                level.append(plsc.sort_key_val(vals if c % 2 == 0 else -vals, chunk_idx[c]))
            while len(level) > 1:
                level = [
                    merge(level[i], level[i + 1], out_inverted=(i // 2) % 2 == 1 and len(level) > 2)
                    for i in range(0, len(level), 2)
                ]
            for c in range(CHUNKS):
                out_v[r, pl.ds(c * L, L)] = zeros16
            idx9 = jnp.where(lane == K, i_splat, level[0][1])
            plsc.store_scatter(out_v, [i_splat, idx9], ninth16, mask=m_sel)

        plsc.parallel_loop(0, P, unroll=2)(row_body)

    in_bufs = [(in_v0, isem0), (in_v1, isem1)]
    out_bufs = [(out_v0, osem0), (out_v1, osem1)]

    def in_slice(ib):
        return pl.ds((wid * 16 + ib) * (2 * P), 2 * P)

    def out_slice(ob):
        return pl.ds((wid * 32 + ob) * P, P)

    pltpu.async_copy(d_hbm.at[in_slice(0)], in_v0, isem0)
    pltpu.async_copy(d_hbm.at[in_slice(1)], in_v1, isem1)

    def pair_body(g, _):
        for p, (in_b, isem) in enumerate(in_bufs):
            ib = 2 * g + p
            pltpu.make_async_copy(d_hbm.at[in_slice(0)], in_b, isem).wait()

            for m, (out_b, osem) in enumerate(out_bufs):
                ob = 2 * ib + m

                @pl.when(ob >= 2)
                def _():
                    pltpu.make_async_copy(out_b, out_hbm.at[out_slice(0)], osem).wait()

                compute_block(in_b, m * P, out_b)
                pltpu.async_copy(out_b, out_hbm.at[out_slice(ob)], osem)

            @pl.when(ib + 2 < 16)
            def _():
                pltpu.async_copy(d_hbm.at[in_slice(ib + 2)], in_b, isem)

        return ()

    lax.fori_loop(0, 8, pair_body, ())
    pltpu.make_async_copy(out_v0, out_hbm.at[out_slice(0)], osem0).wait()
    pltpu.make_async_copy(out_v1, out_hbm.at[out_slice(0)], osem1).wait()


@jax.jit
def _dyn_graph(d2):
    rows = d2.shape[0]
    mesh = plsc.VectorSubcoreMesh(core_axis_name="c", subcore_axis_name="s")
    return pl.kernel(
        _sc_body,
        out_type=jax.ShapeDtypeStruct((rows, P), jnp.float32),
        mesh=mesh,
        scratch_types=[
            pltpu.VMEM((2 * P, P), jnp.float32),
            pltpu.VMEM((2 * P, P), jnp.float32),
            pltpu.VMEM((P, P), jnp.float32),
            pltpu.VMEM((P, P), jnp.float32),
            pltpu.SemaphoreType.DMA,
            pltpu.SemaphoreType.DMA,
            pltpu.SemaphoreType.DMA,
            pltpu.SemaphoreType.DMA,
        ],
        compiler_params=pltpu.CompilerParams(needs_layout_passes=False),
    )(d2)


def kernel(distances):
    B, T, Pa, Pb = distances.shape
    d2 = distances.reshape(B * T * Pa, Pb)
    out = _dyn_graph(d2)
    return out.reshape(B, T, Pa, Pb)


# speedup vs baseline: 1.0163x; 1.0163x over previous
import jax
import jax.numpy as jnp
from jax import lax
from jax.experimental import pallas as pl
from jax.experimental.pallas import tpu as pltpu
from jax.experimental.pallas import tpu_sc as plsc

P = 128
K = 8
L = 16
CHUNKS = P // L
BIG = 3.0e38
IDX_MASK = 0x7F
NINTH = 1.0 / 9.0


def _sc_body(d_hbm, out_hbm, in_v0, in_v1, out_v0, out_v1,
             isem0, isem1, osem0, osem1):
    num_cores = 2
    wid = lax.axis_index("s") * num_cores + lax.axis_index("c")

    lane = lax.iota(jnp.int32, L)
    m_lo = lane < K
    m_sel = lane <= K
    chunk_idx = [lane + L * c for c in range(CHUNKS)]
    zeros16 = jnp.zeros((L,), jnp.float32)
    ninth16 = jnp.full((L,), NINTH, jnp.float32)
    big16 = jnp.full((L,), BIG, jnp.float32)
    hi_mask = jnp.full((L,), ~IDX_MASK, jnp.int32)
    lo_mask = jnp.full((L,), IDX_MASK, jnp.int32)

    def merge(a, b_inv, out_inverted):
        if out_inverted:
            c = jnp.where(m_lo, ~a, b_inv)
        else:
            c = jnp.where(m_lo, a, ~b_inv)
        return lax.sort(c, dimension=0)

    def compute_block(in_v, off, out_v):
        for g in range(CHUNKS):
            diag = lane + L * g
            plsc.store_scatter(in_v, [off + diag, diag], big16)

        def row_body(r):
            i_splat = jnp.full((L,), r, jnp.int32)
            level = []
            for c in range(CHUNKS):
                bits = lax.bitcast_convert_type(in_v[off + r, pl.ds(c * L, L)], jnp.int32)
                key = (bits & hi_mask) | chunk_idx[c]
                level.append(lax.sort(key if c % 2 == 0 else ~key, dimension=0))
            while len(level) > 1:
                level = [
                    merge(level[i], level[i + 1], out_inverted=(i // 2) % 2 == 1 and len(level) > 2)
                    for i in range(0, len(level), 2)
                ]
            for c in range(CHUNKS):
                out_v[r, pl.ds(c * L, L)] = zeros16
            idx9 = jnp.where(lane == K, i_splat, level[0] & lo_mask)
            plsc.store_scatter(out_v, [i_splat, idx9], ninth16, mask=m_sel)

        plsc.parallel_loop(0, P, unroll=2)(row_body)

    in_bufs = [(in_v0, isem0), (in_v1, isem1)]
    out_bufs = [(out_v0, osem0), (out_v1, osem1)]

    def in_slice(ib):
        return pl.ds((wid * 16 + ib) * (2 * P), 2 * P)

    def out_slice(ob):
        return pl.ds((wid * 32 + ob) * P, P)

    pltpu.async_copy(d_hbm.at[in_slice(0)], in_v0, isem0)
    pltpu.async_copy(d_hbm.at[in_slice(1)], in_v1, isem1)

    def pair_body(g, _):
        for p, (in_b, isem) in enumerate(in_bufs):
            ib = 2 * g + p
            pltpu.make_async_copy(d_hbm.at[in_slice(0)], in_b, isem).wait()

            for m, (out_b, osem) in enumerate(out_bufs):
                ob = 2 * ib + m

                @pl.when(ob >= 2)
                def _():
                    pltpu.make_async_copy(out_b, out_hbm.at[out_slice(0)], osem).wait()

                compute_block(in_b, m * P, out_b)
                pltpu.async_copy(out_b, out_hbm.at[out_slice(ob)], osem)

            @pl.when(ib + 2 < 16)
            def _():
                pltpu.async_copy(d_hbm.at[in_slice(ib + 2)], in_b, isem)

        return ()

    lax.fori_loop(0, 8, pair_body, ())
    pltpu.make_async_copy(out_v0, out_hbm.at[out_slice(0)], osem0).wait()
    pltpu.make_async_copy(out_v1, out_hbm.at[out_slice(0)], osem1).wait()


@jax.jit
def _dyn_graph(d2):
    rows = d2.shape[0]
    mesh = plsc.VectorSubcoreMesh(core_axis_name="c", subcore_axis_name="s")
    return pl.kernel(
        _sc_body,
        out_type=jax.ShapeDtypeStruct((rows, P), jnp.float32),
        mesh=mesh,
        scratch_types=[
            pltpu.VMEM((2 * P, P), jnp.float32),
            pltpu.VMEM((2 * P, P), jnp.float32),
            pltpu.VMEM((P, P), jnp.float32),
            pltpu.VMEM((P, P), jnp.float32),
            pltpu.SemaphoreType.DMA,
            pltpu.SemaphoreType.DMA,
            pltpu.SemaphoreType.DMA,
            pltpu.SemaphoreType.DMA,
        ],
        compiler_params=pltpu.CompilerParams(needs_layout_passes=False),
    )(d2)


def kernel(distances):
    B, T, Pa, Pb = distances.shape
    d2 = distances.reshape(B * T * Pa, Pb)
    out = _dyn_graph(d2)
    return out.reshape(B, T, Pa, Pb)
